# baseline (speedup 1.0000x reference)
import functools

import numpy as np

import jax
import jax.numpy as jnp
from jax import lax
from jax.experimental import pallas as pl
from jax.experimental.pallas import tpu as pltpu
from jax.experimental.pallas import tpu_sc as plsc

N = 10000
E = 320000
D_IN = 128
D_EDGE = 4
NC, NS, L = 2, 16, 16
NW = NC * NS
NSUB = 4
CS = NSUB * 128
E_PAD = 327680
EPT = E_PAD // NW
NCHUNK = EPT // CS
EPS = 1e-16
F32 = jnp.float32


def _tc0a_body(x_ref, wl_ref, wr_ref, xl_ref, xr_ref):
    x = x_ref[...]
    z = jnp.zeros((N, 8), F32)
    xl = jnp.dot(x, wl_ref[...], preferred_element_type=F32)
    xr = jnp.dot(x, wr_ref[...], preferred_element_type=F32)
    xl_ref[...] = jnp.concatenate([xl, z], axis=1)
    xr_ref[...] = jnp.concatenate([xr, z], axis=1)


def _tc0a(x, wl, wr):
    return pl.pallas_call(
        _tc0a_body,
        out_shape=(
            jax.ShapeDtypeStruct((N, 16), F32),
            jax.ShapeDtypeStruct((N, 16), F32),
        ),
    )(x, wl, wr)


def _tc0b_body(ea_ref, we_ref, out_ref):
    blk = ea_ref[...].shape[0]
    ea = jnp.dot(ea_ref[...], we_ref[...], preferred_element_type=F32)
    out_ref[...] = jnp.concatenate([ea, jnp.zeros((blk, 8), F32)], axis=1)


def _tc0b(eap, we):
    nblk = E_PAD // 4096
    return pl.pallas_call(
        _tc0b_body,
        grid=(nblk,),
        in_specs=[
            pl.BlockSpec((4096, D_EDGE), lambda i: (i, 0)),
            pl.BlockSpec((D_EDGE, 8), lambda i: (0, 0)),
        ],
        out_specs=pl.BlockSpec((4096, 16), lambda i: (i, 0)),
        out_shape=jax.ShapeDtypeStruct((E_PAD, 16), F32),
    )(eap, we)


def _tc_mid_body(d_in,
                 acc_ref, den_ref, b_ref, wl_ref, wr_ref,
                 xl_ref, xr_ref, denf_ref):
    acc = acc_ref[0] + acc_ref[1]
    den = den_ref[0] + den_ref[1]
    h = acc[:, :d_in] / (den + EPS) + b_ref[...]
    h = jnp.maximum(h, 0.0)
    xl_ref[...] = jnp.dot(h, wl_ref[...], preferred_element_type=F32)
    xr_ref[...] = jnp.dot(h, wr_ref[...], preferred_element_type=F32)
    denf_ref[...] = den


def _tc_mid(d_in, d_out, accp, denp, b, wl, wr):
    return pl.pallas_call(
        functools.partial(_tc_mid_body, d_in),
        out_shape=(
            jax.ShapeDtypeStruct((N, d_out), F32),
            jax.ShapeDtypeStruct((N, d_out), F32),
            jax.ShapeDtypeStruct((N, 1), F32),
        ),
    )(accp, denp, b, wl, wr)


def _tc3_body(acc_ref, den_ref, b_ref, h_ref, denf_ref):
    acc = acc_ref[0] + acc_ref[1]
    den = den_ref[0] + den_ref[1]
    h_ref[...] = acc / (den + EPS) + b_ref[...]
    denf_ref[...] = den


def _tc3(accp, denp, b):
    return pl.pallas_call(
        _tc3_body,
        out_shape=(
            jax.ShapeDtypeStruct((N, 64), F32),
            jax.ShapeDtypeStruct((N, 1), F32),
        ),
    )(accp, denp, b)


def _sc_layer_body(dpad, has_prev,
                   src_hbm, dst_hbm, ea_hbm, xl_hbm, xr_hbm, attb_hbm,
                   crot_hbm, *rest):
    if has_prev:
        (web_hbm, exprev_hbm, denprev_hbm, zacc_hbm, zden_hbm,
         acc_out, den_out, ex_out,
         idx_sa, idx_da,
         xls0, xls1, xls2, xls3, xrd0, xrd1, xrd2, xrd3,
         exb0, exb1, exb2, exb3, attb_v, crot_v, acc_sh, den_sh,
         s0, s1, s2, s3, s4, s5, s6, s7, s8, s9, s10,
         s11, s12, s13, s14, s15, s16, s17, s18, s19, s20, s21, s22,
         web_v, exprev_v, denprev_v) = rest
        ea_v = None
    else:
        (zacc_hbm, zden_hbm,
         acc_out, den_out, ex_out,
         idx_sa, idx_da,
         xls0, xls1, xls2, xls3, xrd0, xrd1, xrd2, xrd3,
         exb0, exb1, exb2, exb3, attb_v, crot_v, acc_sh, den_sh,
         s0, s1, s2, s3, s4, s5, s6, s7, s8, s9, s10,
         s11, s12, s13, s14, s15, s16, s17, s18, s19, s20, s21, s22,
         ea_v) = rest
        web_v = exprev_v = denprev_v = None
    xls = [xls0, xls1, xls2, xls3]
    xrd = [xrd0, xrd1, xrd2, xrd3]
    exb = [exb0, exb1, exb2, exb3]
    gsem = [s0, s1, s2, s3, s4, s5, s6, s7, s8]
    osem = [s9, s10, s11, s12, s13, s14, s15, s16, s17, s18, s19, s20]
    isem = [s21, s22]

    cid = lax.axis_index("c")
    sid = lax.axis_index("s")
    wid = sid * NC + cid

    @pl.when(sid == 0)
    def _zero():
        pltpu.sync_copy(zacc_hbm, acc_sh)
        pltpu.sync_copy(zden_hbm, den_sh)

    pltpu.sync_copy(attb_hbm, attb_v)
    pltpu.sync_copy(crot_hbm, crot_v)
    if has_prev:
        pltpu.sync_copy(web_hbm, web_v)
        pltpu.sync_copy(denprev_hbm, denprev_v)
    plsc.subcore_barrier()

    base0 = wid * EPT
    iota = lax.iota(jnp.int32, L)

    rows_pt = EPT // 128

    def chunk_body(ci, carry):
        base = base0 + ci * CS
        r0 = wid * rows_pt + ci * NSUB
        d_i0 = pltpu.async_copy(src_hbm.at[pl.ds(r0, NSUB)], idx_sa, isem[0])
        d_i1 = pltpu.async_copy(dst_hbm.at[pl.ds(r0, NSUB)], idx_da, isem[1])
        if has_prev:
            d_ea = pltpu.async_copy(exprev_hbm.at[pl.ds(base, CS)], exprev_v,
                                    gsem[8])
        else:
            d_ea = pltpu.async_copy(ea_hbm.at[pl.ds(base, CS)], ea_v, gsem[8])
        d_i0.wait()
        d_i1.wait()
        gl = [pltpu.async_copy(xl_hbm.at[idx_sa.at[b]], xls[b], gsem[b])
              for b in range(NSUB)]
        gr = [pltpu.async_copy(xr_hbm.at[idx_da.at[b]], xrd[b],
                               gsem[4 + b]) for b in range(NSUB)]
        d_ea.wait()

        ws = []
        for b in range(NSUB):
            gl[b].wait()
            gr[b].wait()

            @plsc.parallel_loop(0, 128 // L, unroll=2)
            def group_body(g, b=b):
                row16 = g * L + iota
                if has_prev:
                    di16 = idx_da[b, pl.ds(g * L, L)]
                    d16 = plsc.load_gather(denprev_v, [di16])
                    a16 = exprev_v[pl.ds(b * 128 + g * L, L)] / (d16 + EPS)
                s16 = jnp.zeros((L,), F32)
                for k in range(dpad):
                    colk = crot_v[k]
                    m = (plsc.load_gather(xls[b], [row16, colk])
                         + plsc.load_gather(xrd[b], [row16, colk]))
                    if has_prev:
                        m = m + a16 * web_v[k]
                    else:
                        m = m + plsc.load_gather(
                            ea_v, [b * 128 + row16, colk])
                    s16 = s16 + jnp.maximum(m, 0.2 * m) * attb_v[k]
                ex16 = jnp.exp(s16)
                ge16 = base + b * 128 + g * L + iota
                ex16 = jnp.where(ge16 < E, ex16, 0.0)
                exb[b][pl.ds(g * L, L)] = ex16
                for k in range(dpad):
                    colk = crot_v[k]
                    v = ex16 * plsc.load_gather(xls[b], [row16, colk])
                    plsc.store_scatter(xls[b], [row16, colk], v)

            ws.append(pltpu.async_copy(
                exb[b], ex_out.at[pl.ds(base + b * 128, 128)], osem[b]))
            ws.append(pltpu.async_copy(
                xls[b], acc_sh.at[idx_da.at[b]], osem[4 + b], add=True))
            ws.append(pltpu.async_copy(
                exb[b], den_sh.at[idx_da.at[b]], osem[8 + b], add=True))
        for w in ws:
            w.wait()
        return carry

    lax.fori_loop(0, NCHUNK, chunk_body, 0)

    plsc.subcore_barrier()

    @pl.when(sid == 0)
    def _flush():
        pltpu.sync_copy(acc_sh, acc_out.at[cid])
        pltpu.sync_copy(den_sh, den_out.at[cid])


def _sc_layer(dpad, has_prev):
    mesh = plsc.VectorSubcoreMesh(core_axis_name="c", subcore_axis_name="s")
    scratch = (
        [pltpu.VMEM((NSUB, 128), jnp.int32) for _ in range(2)]
        + [pltpu.VMEM((128, dpad), F32) for _ in range(2 * NSUB)]
        + [pltpu.VMEM((128,), F32) for _ in range(NSUB)]
        + [
            pltpu.VMEM((dpad, L), F32),
            pltpu.VMEM((dpad, L), jnp.int32),
            pltpu.VMEM_SHARED((N, dpad), F32),
            pltpu.VMEM_SHARED((N,), F32),
        ]
        + [pltpu.SemaphoreType.DMA for _ in range(23)]
    )
    if has_prev:
        scratch += [
            pltpu.VMEM((dpad, L), F32),
            pltpu.VMEM((CS,), F32),
            pltpu.VMEM((N,), F32),
        ]
    else:
        scratch += [pltpu.VMEM((CS, 16), F32)]

    return pl.kernel(
        functools.partial(_sc_layer_body, dpad, has_prev),
        out_type=(
            jax.ShapeDtypeStruct((NC, N, dpad), F32),
            jax.ShapeDtypeStruct((NC, N), F32),
            jax.ShapeDtypeStruct((E_PAD,), F32),
        ),
        mesh=mesh,
        compiler_params=pltpu.CompilerParams(
            needs_layout_passes=False, use_tc_tiling_on_sc=False),
        scratch_types=scratch,
    )


def _sc_alpha_body(dst_hbm, ex_hbm, den_hbm, a_out, idx_d, exv, av, den_v):
    cid = lax.axis_index("c")
    sid = lax.axis_index("s")
    wid = sid * NC + cid
    pltpu.sync_copy(den_hbm, den_v)
    base0 = wid * EPT

    def chunk_body(ci, carry):
        base = base0 + ci * CS
        pltpu.sync_copy(dst_hbm.at[pl.ds(base, CS)], idx_d)
        pltpu.sync_copy(ex_hbm.at[pl.ds(base, CS)], exv)

        def group_body(g, gcarry):
            di16 = idx_d[pl.ds(g * L, L)]
            d16 = plsc.load_gather(den_v, [di16])
            av[pl.ds(g * L, L)] = exv[pl.ds(g * L, L)] / (d16 + EPS)
            return gcarry

        lax.fori_loop(0, CS // L, group_body, 0)
        pltpu.sync_copy(av, a_out.at[pl.ds(base, CS)])
        return carry

    lax.fori_loop(0, NCHUNK, chunk_body, 0)


def _sc_alpha():
    mesh = plsc.VectorSubcoreMesh(core_axis_name="c", subcore_axis_name="s")
    return pl.kernel(
        _sc_alpha_body,
        out_type=jax.ShapeDtypeStruct((E_PAD,), F32),
        mesh=mesh,
        compiler_params=pltpu.CompilerParams(
            needs_layout_passes=False, use_tc_tiling_on_sc=False),
        scratch_types=[
            pltpu.VMEM((CS,), jnp.int32),
            pltpu.VMEM((CS,), F32),
            pltpu.VMEM((CS,), F32),
            pltpu.VMEM((N,), F32),
        ],
    )


def kernel(x, edge_index, edge_attr,
           Wl1, Wr1, att1, b1, We1,
           Wl2, Wr2, att2, b2, We2,
           Wl3, Wr3, att3, b3, We3):
    src = edge_index[0].astype(jnp.int32)
    dst = edge_index[1].astype(jnp.int32)
    srcp = jnp.pad(src, (0, E_PAD - E)).reshape(E_PAD // 128, 128)
    dstp = jnp.pad(dst, (0, E_PAD - E)).reshape(E_PAD // 128, 128)
    eap = jnp.pad(edge_attr, ((0, E_PAD - E), (0, 0)))

    zacc16 = jnp.zeros((N, 16), F32)
    zacc64 = jnp.zeros((N, 64), F32)
    zden = jnp.zeros((N,), F32)

    rot16 = jnp.asarray((np.arange(16)[:, None] + np.arange(L)[None, :]) % 16,
                        jnp.int32)
    rot64 = jnp.asarray((np.arange(64)[:, None] + np.arange(L)[None, :]) % 64,
                        jnp.int32)
    att1p = jnp.concatenate([att1, jnp.zeros((8,), F32)])
    attb1 = att1p[rot16]
    attb2 = att2[rot16]
    web2 = We2.reshape(16)[rot16]
    attb3 = att3[rot64]
    web3 = We3.reshape(64)[rot64]

    xl1, xr1 = _tc0a(x, Wl1, Wr1)
    ea1 = _tc0b(eap, We1)
    accp1, denp1, ex1 = _sc_layer(16, False)(
        srcp, dstp, ea1, xl1, xr1, attb1, rot16, zacc16, zden)

    xl2, xr2, den1f = _tc_mid(
        8, 16, accp1, denp1[:, :, None], b1.reshape(1, 8), Wl2, Wr2)
    accp2, denp2, ex2 = _sc_layer(16, True)(
        srcp, dstp, ex1, xl2, xr2, attb2, rot16, web2, ex1, den1f.reshape(N),
        zacc16, zden)

    xl3, xr3, den2f = _tc_mid(
        16, 64, accp2, denp2[:, :, None], b2.reshape(1, 16), Wl3, Wr3)
    accp3, denp3, ex3 = _sc_layer(64, True)(
        srcp, dstp, ex2, xl3, xr3, attb3, rot64, web3, ex2, den2f.reshape(N),
        zacc64, zden)

    h, den3f = _tc3(accp3, denp3[:, :, None], b3.reshape(1, 64))
    a3p = _sc_alpha()(dstp.reshape(E_PAD), ex3, den3f.reshape(N))
    return (h, edge_index, a3p[:E])


# speedup vs baseline: 1.1911x; 1.1911x over previous
import functools

import numpy as np

import jax
import jax.numpy as jnp
from jax import lax
from jax.experimental import pallas as pl
from jax.experimental.pallas import tpu as pltpu
from jax.experimental.pallas import tpu_sc as plsc

N = 10000
E = 320000
D_IN = 128
D_EDGE = 4
NC, NS, L = 2, 16, 16
NW = NC * NS
NSUB = 4
CS = NSUB * 128
E_PAD = 327680
EPT = E_PAD // NW
NCHUNK = EPT // CS
EPS = 1e-16
F32 = jnp.float32


def _tc0a_body(x_ref, wl_ref, wr_ref, xl_ref, xr_ref):
    x = x_ref[...]
    z = jnp.zeros((N, 8), F32)
    xl = jnp.dot(x, wl_ref[...], preferred_element_type=F32)
    xr = jnp.dot(x, wr_ref[...], preferred_element_type=F32)
    xl_ref[...] = jnp.concatenate([xl, z], axis=1)
    xr_ref[...] = jnp.concatenate([xr, z], axis=1)


def _tc0a(x, wl, wr):
    return pl.pallas_call(
        _tc0a_body,
        out_shape=(
            jax.ShapeDtypeStruct((N, 16), F32),
            jax.ShapeDtypeStruct((N, 16), F32),
        ),
    )(x, wl, wr)


def _tc0b_body(ea_ref, we_ref, out_ref):
    blk = ea_ref[...].shape[0]
    ea = jnp.dot(ea_ref[...], we_ref[...], preferred_element_type=F32)
    out_ref[...] = jnp.concatenate([ea, jnp.zeros((blk, 8), F32)], axis=1)


def _tc0b(eap, we):
    nblk = E_PAD // 4096
    return pl.pallas_call(
        _tc0b_body,
        grid=(nblk,),
        in_specs=[
            pl.BlockSpec((4096, D_EDGE), lambda i: (i, 0)),
            pl.BlockSpec((D_EDGE, 8), lambda i: (0, 0)),
        ],
        out_specs=pl.BlockSpec((4096, 16), lambda i: (i, 0)),
        out_shape=jax.ShapeDtypeStruct((E_PAD, 16), F32),
    )(eap, we)


def _tc_mid_body(d_in,
                 acc_ref, den_ref, b_ref, wl_ref, wr_ref,
                 xl_ref, xr_ref, denf_ref):
    acc = acc_ref[0] + acc_ref[1]
    den = den_ref[0] + den_ref[1]
    h = acc[:, :d_in] / (den + EPS) + b_ref[...]
    h = jnp.maximum(h, 0.0)
    xl_ref[...] = jnp.dot(h, wl_ref[...], preferred_element_type=F32)
    xr_ref[...] = jnp.dot(h, wr_ref[...], preferred_element_type=F32)
    denf_ref[...] = den


def _tc_mid(d_in, d_out, accp, denp, b, wl, wr):
    return pl.pallas_call(
        functools.partial(_tc_mid_body, d_in),
        out_shape=(
            jax.ShapeDtypeStruct((N, d_out), F32),
            jax.ShapeDtypeStruct((N, d_out), F32),
            jax.ShapeDtypeStruct((N, 1), F32),
        ),
    )(accp, denp, b, wl, wr)


def _tc3_body(acc_ref, den_ref, b_ref, h_ref, denf_ref):
    acc = acc_ref[0] + acc_ref[1]
    den = den_ref[0] + den_ref[1]
    h_ref[...] = acc / (den + EPS) + b_ref[...]
    denf_ref[...] = den


def _tc3(accp, denp, b):
    return pl.pallas_call(
        _tc3_body,
        out_shape=(
            jax.ShapeDtypeStruct((N, 64), F32),
            jax.ShapeDtypeStruct((N, 1), F32),
        ),
    )(accp, denp, b)


def _sc_layer_body(dpad, has_prev,
                   src_hbm, dst_hbm, ea_hbm, xl_hbm, xr_hbm, attb_hbm,
                   crot_hbm, *rest):
    if has_prev:
        (web_hbm, exprev_hbm, denprev_hbm, zacc_hbm, zden_hbm,
         acc_out, den_out, ex_out,
         idx_sa, idx_da,
         xls0, xls1, xls2, xls3, xrd0, xrd1, xrd2, xrd3,
         exb0, exb1, exb2, exb3, attb_v, crot_v, acc_sh, den_sh,
         s0, s1, s2, s3, s4, s5, s6, s7, s8, s9, s10,
         s11, s12, s13, s14, s15, s16, s17, s18, s19, s20, s21, s22,
         web_v, exprev_v, denprev_v) = rest
        ea_v = None
    else:
        (zacc_hbm, zden_hbm,
         acc_out, den_out, ex_out,
         idx_sa, idx_da,
         xls0, xls1, xls2, xls3, xrd0, xrd1, xrd2, xrd3,
         exb0, exb1, exb2, exb3, attb_v, crot_v, acc_sh, den_sh,
         s0, s1, s2, s3, s4, s5, s6, s7, s8, s9, s10,
         s11, s12, s13, s14, s15, s16, s17, s18, s19, s20, s21, s22,
         ea_v) = rest
        web_v = exprev_v = denprev_v = None
    xls = [xls0, xls1, xls2, xls3]
    xrd = [xrd0, xrd1, xrd2, xrd3]
    exb = [exb0, exb1, exb2, exb3]
    gsem = [s0, s1, s2, s3, s4, s5, s6, s7, s8]
    osem = [s9, s10, s11, s12, s13, s14, s15, s16, s17, s18, s19, s20]
    isem = [s21, s22]

    cid = lax.axis_index("c")
    sid = lax.axis_index("s")
    wid = sid * NC + cid

    @pl.when(sid == 0)
    def _zero():
        pltpu.sync_copy(zacc_hbm, acc_sh)
        pltpu.sync_copy(zden_hbm, den_sh)

    pltpu.sync_copy(attb_hbm, attb_v)
    pltpu.sync_copy(crot_hbm, crot_v)
    if has_prev:
        pltpu.sync_copy(web_hbm, web_v)
        pltpu.sync_copy(denprev_hbm, denprev_v)
    plsc.subcore_barrier()

    base0 = wid * EPT
    iota = lax.iota(jnp.int32, L)

    rows_pt = EPT // 128

    def chunk_body(ci, carry):
        base = base0 + ci * CS
        r0 = wid * rows_pt + ci * NSUB
        d_i0 = pltpu.async_copy(src_hbm.at[pl.ds(r0, NSUB)], idx_sa, isem[0])
        d_i1 = pltpu.async_copy(dst_hbm.at[pl.ds(r0, NSUB)], idx_da, isem[1])
        if has_prev:
            d_ea = pltpu.async_copy(exprev_hbm.at[pl.ds(base, CS)], exprev_v,
                                    gsem[8])
        else:
            d_ea = pltpu.async_copy(ea_hbm.at[pl.ds(base, CS)], ea_v, gsem[8])
        d_i0.wait()
        d_i1.wait()
        gl = [pltpu.async_copy(xl_hbm.at[idx_sa.at[b]], xls[b], gsem[b])
              for b in range(NSUB)]
        gr = [pltpu.async_copy(xr_hbm.at[idx_da.at[b]], xrd[b],
                               gsem[4 + b]) for b in range(NSUB)]
        d_ea.wait()

        ws = []
        for b in range(NSUB):
            gl[b].wait()
            gr[b].wait()

            def group_body(g, gcarry, b=b):
                row16 = g * L + iota
                if has_prev:
                    di16 = idx_da[b, pl.ds(g * L, L)]
                    d16 = plsc.load_gather(denprev_v, [di16])
                    a16 = exprev_v[pl.ds(b * 128 + g * L, L)] / (d16 + EPS)
                part = [jnp.zeros((L,), F32) for _ in range(4)]
                for k in range(dpad):
                    colk = crot_v[k]
                    m = (plsc.load_gather(xls[b], [row16, colk])
                         + plsc.load_gather(xrd[b], [row16, colk]))
                    if has_prev:
                        m = m + a16 * web_v[k]
                    else:
                        m = m + plsc.load_gather(
                            ea_v, [b * 128 + row16, colk])
                    part[k % 4] = part[k % 4] + jnp.maximum(m, 0.2 * m) * attb_v[k]
                ex16 = jnp.exp((part[0] + part[1]) + (part[2] + part[3]))
                ge16 = base + b * 128 + g * L + iota
                ex16 = jnp.where(ge16 < E, ex16, 0.0)
                exb[b][pl.ds(g * L, L)] = ex16
                for k in range(dpad):
                    colk = crot_v[k]
                    v = ex16 * plsc.load_gather(xls[b], [row16, colk])
                    plsc.store_scatter(xls[b], [row16, colk], v)
                return gcarry

            lax.fori_loop(0, 128 // L, group_body, 0)

            ws.append(pltpu.async_copy(
                exb[b], ex_out.at[pl.ds(base + b * 128, 128)], osem[b]))
            ws.append(pltpu.async_copy(
                xls[b], acc_sh.at[idx_da.at[b]], osem[4 + b], add=True))
            ws.append(pltpu.async_copy(
                exb[b], den_sh.at[idx_da.at[b]], osem[8 + b], add=True))
        for w in ws:
            w.wait()
        return carry

    lax.fori_loop(0, NCHUNK, chunk_body, 0)

    plsc.subcore_barrier()

    @pl.when(sid == 0)
    def _flush():
        pltpu.sync_copy(acc_sh, acc_out.at[cid])
        pltpu.sync_copy(den_sh, den_out.at[cid])


def _sc_layer(dpad, has_prev):
    mesh = plsc.VectorSubcoreMesh(core_axis_name="c", subcore_axis_name="s")
    scratch = (
        [pltpu.VMEM((NSUB, 128), jnp.int32) for _ in range(2)]
        + [pltpu.VMEM((128, dpad), F32) for _ in range(2 * NSUB)]
        + [pltpu.VMEM((128,), F32) for _ in range(NSUB)]
        + [
            pltpu.VMEM((dpad, L), F32),
            pltpu.VMEM((dpad, L), jnp.int32),
            pltpu.VMEM_SHARED((N, dpad), F32),
            pltpu.VMEM_SHARED((N,), F32),
        ]
        + [pltpu.SemaphoreType.DMA for _ in range(23)]
    )
    if has_prev:
        scratch += [
            pltpu.VMEM((dpad, L), F32),
            pltpu.VMEM((CS,), F32),
            pltpu.VMEM((N,), F32),
        ]
    else:
        scratch += [pltpu.VMEM((CS, 16), F32)]

    return pl.kernel(
        functools.partial(_sc_layer_body, dpad, has_prev),
        out_type=(
            jax.ShapeDtypeStruct((NC, N, dpad), F32),
            jax.ShapeDtypeStruct((NC, N), F32),
            jax.ShapeDtypeStruct((E_PAD,), F32),
        ),
        mesh=mesh,
        compiler_params=pltpu.CompilerParams(
            needs_layout_passes=False, use_tc_tiling_on_sc=False),
        scratch_types=scratch,
    )


def _sc_alpha_body(dst_hbm, ex_hbm, den_hbm, a_out, idx_d, exv, av, den_v):
    cid = lax.axis_index("c")
    sid = lax.axis_index("s")
    wid = sid * NC + cid
    pltpu.sync_copy(den_hbm, den_v)
    base0 = wid * EPT

    def chunk_body(ci, carry):
        base = base0 + ci * CS
        pltpu.sync_copy(dst_hbm.at[pl.ds(base, CS)], idx_d)
        pltpu.sync_copy(ex_hbm.at[pl.ds(base, CS)], exv)

        def group_body(g, gcarry):
            di16 = idx_d[pl.ds(g * L, L)]
            d16 = plsc.load_gather(den_v, [di16])
            av[pl.ds(g * L, L)] = exv[pl.ds(g * L, L)] / (d16 + EPS)
            return gcarry

        lax.fori_loop(0, CS // L, group_body, 0)
        pltpu.sync_copy(av, a_out.at[pl.ds(base, CS)])
        return carry

    lax.fori_loop(0, NCHUNK, chunk_body, 0)


def _sc_alpha():
    mesh = plsc.VectorSubcoreMesh(core_axis_name="c", subcore_axis_name="s")
    return pl.kernel(
        _sc_alpha_body,
        out_type=jax.ShapeDtypeStruct((E_PAD,), F32),
        mesh=mesh,
        compiler_params=pltpu.CompilerParams(
            needs_layout_passes=False, use_tc_tiling_on_sc=False),
        scratch_types=[
            pltpu.VMEM((CS,), jnp.int32),
            pltpu.VMEM((CS,), F32),
            pltpu.VMEM((CS,), F32),
            pltpu.VMEM((N,), F32),
        ],
    )


def kernel(x, edge_index, edge_attr,
           Wl1, Wr1, att1, b1, We1,
           Wl2, Wr2, att2, b2, We2,
           Wl3, Wr3, att3, b3, We3):
    src = edge_index[0].astype(jnp.int32)
    dst = edge_index[1].astype(jnp.int32)
    srcp = jnp.pad(src, (0, E_PAD - E)).reshape(E_PAD // 128, 128)
    dstp = jnp.pad(dst, (0, E_PAD - E)).reshape(E_PAD // 128, 128)
    eap = jnp.pad(edge_attr, ((0, E_PAD - E), (0, 0)))

    zacc16 = jnp.zeros((N, 16), F32)
    zacc64 = jnp.zeros((N, 64), F32)
    zden = jnp.zeros((N,), F32)

    rot16 = jnp.asarray((np.arange(16)[:, None] + np.arange(L)[None, :]) % 16,
                        jnp.int32)
    rot64 = jnp.asarray((np.arange(64)[:, None] + np.arange(L)[None, :]) % 64,
                        jnp.int32)
    att1p = jnp.concatenate([att1, jnp.zeros((8,), F32)])
    attb1 = att1p[rot16]
    attb2 = att2[rot16]
    web2 = We2.reshape(16)[rot16]
    attb3 = att3[rot64]
    web3 = We3.reshape(64)[rot64]

    xl1, xr1 = _tc0a(x, Wl1, Wr1)
    ea1 = _tc0b(eap, We1)
    accp1, denp1, ex1 = _sc_layer(16, False)(
        srcp, dstp, ea1, xl1, xr1, attb1, rot16, zacc16, zden)

    xl2, xr2, den1f = _tc_mid(
        8, 16, accp1, denp1[:, :, None], b1.reshape(1, 8), Wl2, Wr2)
    accp2, denp2, ex2 = _sc_layer(16, True)(
        srcp, dstp, ex1, xl2, xr2, attb2, rot16, web2, ex1, den1f.reshape(N),
        zacc16, zden)

    xl3, xr3, den2f = _tc_mid(
        16, 64, accp2, denp2[:, :, None], b2.reshape(1, 16), Wl3, Wr3)
    accp3, denp3, ex3 = _sc_layer(64, True)(
        srcp, dstp, ex2, xl3, xr3, attb3, rot64, web3, ex2, den2f.reshape(N),
        zacc64, zden)

    h, den3f = _tc3(accp3, denp3[:, :, None], b3.reshape(1, 64))
    a3p = _sc_alpha()(dstp.reshape(E_PAD), ex3, den3f.reshape(N))
    return (h, edge_index, a3p[:E])


# speedup vs baseline: 1.5139x; 1.2710x over previous
import functools

import numpy as np

import jax
import jax.numpy as jnp
from jax import lax
from jax.experimental import pallas as pl
from jax.experimental.pallas import tpu as pltpu
from jax.experimental.pallas import tpu_sc as plsc

N = 10000
E = 320000
D_IN = 128
D_EDGE = 4
NC, NS, L = 2, 16, 16
NW = NC * NS
NSUB = 4
CS = NSUB * 128
E_PAD = 327680
EPT = E_PAD // NW
NCHUNK = EPT // CS
EPS = 1e-16
F32 = jnp.float32


def _tc0a_body(x_ref, wl_ref, wr_ref, xl_ref, xr_ref):
    x = x_ref[...]
    z = jnp.zeros((N, 8), F32)
    xl = jnp.dot(x, wl_ref[...], preferred_element_type=F32)
    xr = jnp.dot(x, wr_ref[...], preferred_element_type=F32)
    xl_ref[...] = jnp.concatenate([xl, z], axis=1)
    xr_ref[...] = jnp.concatenate([xr, z], axis=1)


def _tc0a(x, wl, wr):
    return pl.pallas_call(
        _tc0a_body,
        out_shape=(
            jax.ShapeDtypeStruct((N, 16), F32),
            jax.ShapeDtypeStruct((N, 16), F32),
        ),
    )(x, wl, wr)


def _tc0b_body(ea_ref, we_ref, out_ref):
    blk = ea_ref[...].shape[0]
    ea = jnp.dot(ea_ref[...], we_ref[...], preferred_element_type=F32)
    out_ref[...] = jnp.concatenate([ea, jnp.zeros((blk, 8), F32)], axis=1)


def _tc0b(eap, we):
    nblk = E_PAD // 4096
    return pl.pallas_call(
        _tc0b_body,
        grid=(nblk,),
        in_specs=[
            pl.BlockSpec((4096, D_EDGE), lambda i: (i, 0)),
            pl.BlockSpec((D_EDGE, 8), lambda i: (0, 0)),
        ],
        out_specs=pl.BlockSpec((4096, 16), lambda i: (i, 0)),
        out_shape=jax.ShapeDtypeStruct((E_PAD, 16), F32),
    )(eap, we)


def _tc_mid_body(d_in,
                 acc_ref, den_ref, b_ref, wl_ref, wr_ref,
                 xl_ref, xr_ref, denf_ref):
    acc = acc_ref[0] + acc_ref[1]
    den = den_ref[0] + den_ref[1]
    h = acc[:, :d_in] / (den + EPS) + b_ref[...]
    h = jnp.maximum(h, 0.0)
    xl_ref[...] = jnp.dot(h, wl_ref[...], preferred_element_type=F32)
    xr_ref[...] = jnp.dot(h, wr_ref[...], preferred_element_type=F32)
    denf_ref[...] = den


def _tc_mid(d_in, d_out, accp, denp, b, wl, wr):
    return pl.pallas_call(
        functools.partial(_tc_mid_body, d_in),
        out_shape=(
            jax.ShapeDtypeStruct((N, d_out), F32),
            jax.ShapeDtypeStruct((N, d_out), F32),
            jax.ShapeDtypeStruct((N, 1), F32),
        ),
    )(accp, denp, b, wl, wr)


def _tc3_body(acc_ref, den_ref, b_ref, h_ref, denf_ref):
    acc = acc_ref[0] + acc_ref[1]
    den = den_ref[0] + den_ref[1]
    h_ref[...] = acc / (den + EPS) + b_ref[...]
    denf_ref[...] = den


def _tc3(accp, denp, b):
    return pl.pallas_call(
        _tc3_body,
        out_shape=(
            jax.ShapeDtypeStruct((N, 64), F32),
            jax.ShapeDtypeStruct((N, 1), F32),
        ),
    )(accp, denp, b)


def _sc_layer_body(dpad, has_prev,
                   src_hbm, dst_hbm, ea_hbm, xl_hbm, xr_hbm, attb_hbm,
                   crot_hbm, *rest):
    if has_prev:
        (web_hbm, exprev_hbm, denprev_hbm, zacc_hbm, zden_hbm,
         acc_out, den_out, ex_out,
         idx_sa, idx_da,
         xls0, xls1, xls2, xls3, xrd0, xrd1, xrd2, xrd3,
         exb0, exb1, exb2, exb3, attb_v, crot_v, acc_sh, den_sh,
         s0, s1, s2, s3, s4, s5, s6, s7, s8, s9, s10,
         s11, s12, s13, s14, s15, s16, s17, s18, s19, s20, s21, s22,
         web_v, exprev_v, denprev_v) = rest
        ea_v = None
    else:
        (zacc_hbm, zden_hbm,
         acc_out, den_out, ex_out,
         idx_sa, idx_da,
         xls0, xls1, xls2, xls3, xrd0, xrd1, xrd2, xrd3,
         exb0, exb1, exb2, exb3, attb_v, crot_v, acc_sh, den_sh,
         s0, s1, s2, s3, s4, s5, s6, s7, s8, s9, s10,
         s11, s12, s13, s14, s15, s16, s17, s18, s19, s20, s21, s22,
         ea_v) = rest
        web_v = exprev_v = denprev_v = None
    xls = [xls0, xls1, xls2, xls3]
    xrd = [xrd0, xrd1, xrd2, xrd3]
    exb = [exb0, exb1, exb2, exb3]
    gsem = [s0, s1, s2, s3, s4, s5, s6, s7, s8]
    osem = [s9, s10, s11, s12, s13, s14, s15, s16, s17, s18, s19, s20]
    isem = [s21, s22]

    cid = lax.axis_index("c")
    sid = lax.axis_index("s")
    wid = sid * NC + cid

    @pl.when(sid == 0)
    def _zero():
        pltpu.sync_copy(zacc_hbm, acc_sh)
        pltpu.sync_copy(zden_hbm, den_sh)

    pltpu.sync_copy(attb_hbm, attb_v)
    pltpu.sync_copy(crot_hbm, crot_v)
    if has_prev:
        pltpu.sync_copy(web_hbm, web_v)
        pltpu.sync_copy(denprev_hbm, denprev_v)
    plsc.subcore_barrier()

    base0 = wid * EPT
    iota = lax.iota(jnp.int32, L)

    rows_pt = EPT // 128

    def chunk_body(ci, carry):
        base = base0 + ci * CS
        r0 = wid * rows_pt + ci * NSUB
        d_i0 = pltpu.async_copy(src_hbm.at[pl.ds(r0, NSUB)], idx_sa, isem[0])
        d_i1 = pltpu.async_copy(dst_hbm.at[pl.ds(r0, NSUB)], idx_da, isem[1])
        if has_prev:
            d_ea = pltpu.async_copy(exprev_hbm.at[pl.ds(base, CS)], exprev_v,
                                    gsem[8])
        else:
            d_ea = pltpu.async_copy(ea_hbm.at[pl.ds(base, CS)], ea_v, gsem[8])
        d_i0.wait()
        d_i1.wait()
        gl = [pltpu.async_copy(xl_hbm.at[idx_sa.at[b]], xls[b], gsem[b])
              for b in range(NSUB)]
        gr = [pltpu.async_copy(xr_hbm.at[idx_da.at[b]], xrd[b],
                               gsem[4 + b]) for b in range(NSUB)]
        d_ea.wait()

        ws = []
        for b in range(NSUB):
            gl[b].wait()
            gr[b].wait()

            def group_body(g, gcarry, b=b):
                row16 = g * L + iota
                if has_prev:
                    di16 = idx_da[b, pl.ds(g * L, L)]
                    d16 = plsc.load_gather(denprev_v, [di16])
                    a16 = exprev_v[pl.ds(b * 128 + g * L, L)] / (d16 + EPS)
                s16 = jnp.zeros((L,), F32)
                for k in range(dpad):
                    colk = (iota + k) & (dpad - 1)
                    m = (plsc.load_gather(xls[b], [row16, colk])
                         + plsc.load_gather(xrd[b], [row16, colk]))
                    if has_prev:
                        m = m + a16 * web_v[k]
                    else:
                        m = m + plsc.load_gather(
                            ea_v, [b * 128 + row16, colk])
                    s16 = s16 + jnp.maximum(m, 0.2 * m) * attb_v[k]
                ex16 = jnp.exp(s16)
                ge16 = base + b * 128 + g * L + iota
                ex16 = jnp.where(ge16 < E, ex16, 0.0)
                exb[b][pl.ds(g * L, L)] = ex16
                for k in range(dpad):
                    colk = (iota + k) & (dpad - 1)
                    v = ex16 * plsc.load_gather(xls[b], [row16, colk])
                    plsc.store_scatter(xls[b], [row16, colk], v)
                return gcarry

            lax.fori_loop(0, 128 // L, group_body, 0)

            ws.append(pltpu.async_copy(
                exb[b], ex_out.at[pl.ds(base + b * 128, 128)], osem[b]))
            ws.append(pltpu.async_copy(
                xls[b], acc_sh.at[idx_da.at[b]], osem[4 + b], add=True))
            ws.append(pltpu.async_copy(
                exb[b], den_sh.at[idx_da.at[b]], osem[8 + b], add=True))
        for w in ws:
            w.wait()
        return carry

    lax.fori_loop(0, NCHUNK, chunk_body, 0)

    plsc.subcore_barrier()

    @pl.when(sid == 0)
    def _flush():
        pltpu.sync_copy(acc_sh, acc_out.at[cid])
        pltpu.sync_copy(den_sh, den_out.at[cid])


def _sc_layer(dpad, has_prev):
    mesh = plsc.VectorSubcoreMesh(core_axis_name="c", subcore_axis_name="s")
    scratch = (
        [pltpu.VMEM((NSUB, 128), jnp.int32) for _ in range(2)]
        + [pltpu.VMEM((128, dpad), F32) for _ in range(2 * NSUB)]
        + [pltpu.VMEM((128,), F32) for _ in range(NSUB)]
        + [
            pltpu.VMEM((dpad, L), F32),
            pltpu.VMEM((dpad, L), jnp.int32),
            pltpu.VMEM_SHARED((N, dpad), F32),
            pltpu.VMEM_SHARED((N,), F32),
        ]
        + [pltpu.SemaphoreType.DMA for _ in range(23)]
    )
    if has_prev:
        scratch += [
            pltpu.VMEM((dpad, L), F32),
            pltpu.VMEM((CS,), F32),
            pltpu.VMEM((N,), F32),
        ]
    else:
        scratch += [pltpu.VMEM((CS, 16), F32)]

    return pl.kernel(
        functools.partial(_sc_layer_body, dpad, has_prev),
        out_type=(
            jax.ShapeDtypeStruct((NC, N, dpad), F32),
            jax.ShapeDtypeStruct((NC, N), F32),
            jax.ShapeDtypeStruct((E_PAD,), F32),
        ),
        mesh=mesh,
        compiler_params=pltpu.CompilerParams(
            needs_layout_passes=False, use_tc_tiling_on_sc=False),
        scratch_types=scratch,
    )


def _sc_alpha_body(dst_hbm, ex_hbm, den_hbm, a_out, idx_d, exv, av, den_v):
    cid = lax.axis_index("c")
    sid = lax.axis_index("s")
    wid = sid * NC + cid
    pltpu.sync_copy(den_hbm, den_v)
    base0 = wid * EPT

    def chunk_body(ci, carry):
        base = base0 + ci * CS
        pltpu.sync_copy(dst_hbm.at[pl.ds(base, CS)], idx_d)
        pltpu.sync_copy(ex_hbm.at[pl.ds(base, CS)], exv)

        def group_body(g, gcarry):
            di16 = idx_d[pl.ds(g * L, L)]
            d16 = plsc.load_gather(den_v, [di16])
            av[pl.ds(g * L, L)] = exv[pl.ds(g * L, L)] / (d16 + EPS)
            return gcarry

        lax.fori_loop(0, CS // L, group_body, 0)
        pltpu.sync_copy(av, a_out.at[pl.ds(base, CS)])
        return carry

    lax.fori_loop(0, NCHUNK, chunk_body, 0)


def _sc_alpha():
    mesh = plsc.VectorSubcoreMesh(core_axis_name="c", subcore_axis_name="s")
    return pl.kernel(
        _sc_alpha_body,
        out_type=jax.ShapeDtypeStruct((E_PAD,), F32),
        mesh=mesh,
        compiler_params=pltpu.CompilerParams(
            needs_layout_passes=False, use_tc_tiling_on_sc=False),
        scratch_types=[
            pltpu.VMEM((CS,), jnp.int32),
            pltpu.VMEM((CS,), F32),
            pltpu.VMEM((CS,), F32),
            pltpu.VMEM((N,), F32),
        ],
    )


def kernel(x, edge_index, edge_attr,
           Wl1, Wr1, att1, b1, We1,
           Wl2, Wr2, att2, b2, We2,
           Wl3, Wr3, att3, b3, We3):
    src = edge_index[0].astype(jnp.int32)
    dst = edge_index[1].astype(jnp.int32)
    srcp = jnp.pad(src, (0, E_PAD - E)).reshape(E_PAD // 128, 128)
    dstp = jnp.pad(dst, (0, E_PAD - E)).reshape(E_PAD // 128, 128)
    eap = jnp.pad(edge_attr, ((0, E_PAD - E), (0, 0)))

    zacc16 = jnp.zeros((N, 16), F32)
    zacc64 = jnp.zeros((N, 64), F32)
    zden = jnp.zeros((N,), F32)

    rot16 = jnp.asarray((np.arange(16)[:, None] + np.arange(L)[None, :]) % 16,
                        jnp.int32)
    rot64 = jnp.asarray((np.arange(64)[:, None] + np.arange(L)[None, :]) % 64,
                        jnp.int32)
    att1p = jnp.concatenate([att1, jnp.zeros((8,), F32)])
    attb1 = att1p[rot16]
    attb2 = att2[rot16]
    web2 = We2.reshape(16)[rot16]
    attb3 = att3[rot64]
    web3 = We3.reshape(64)[rot64]

    xl1, xr1 = _tc0a(x, Wl1, Wr1)
    ea1 = _tc0b(eap, We1)
    accp1, denp1, ex1 = _sc_layer(16, False)(
        srcp, dstp, ea1, xl1, xr1, attb1, rot16, zacc16, zden)

    xl2, xr2, den1f = _tc_mid(
        8, 16, accp1, denp1[:, :, None], b1.reshape(1, 8), Wl2, Wr2)
    accp2, denp2, ex2 = _sc_layer(16, True)(
        srcp, dstp, ex1, xl2, xr2, attb2, rot16, web2, ex1, den1f.reshape(N),
        zacc16, zden)

    xl3, xr3, den2f = _tc_mid(
        16, 64, accp2, denp2[:, :, None], b2.reshape(1, 16), Wl3, Wr3)
    accp3, denp3, ex3 = _sc_layer(64, True)(
        srcp, dstp, ex2, xl3, xr3, attb3, rot64, web3, ex2, den2f.reshape(N),
        zacc64, zden)

    h, den3f = _tc3(accp3, denp3[:, :, None], b3.reshape(1, 64))
    a3p = _sc_alpha()(dstp.reshape(E_PAD), ex3, den3f.reshape(N))
    return (h, edge_index, a3p[:E])
